# initial kernel scaffold (unmeasured)
import jax
import jax.numpy as jnp
from jax import lax
from jax.experimental import pallas as pl
from jax.experimental.pallas import tpu as pltpu

N_DEV = 4


def _ag_body(x_ref, out_ref, copy_sem, send_sems, recv_sems):
    my = lax.axis_index("i")
    left = (my - 1) % N_DEV
    right = (my + 1) % N_DEV
    m_per = x_ref.shape[0]

    barrier_sem = pltpu.get_barrier_semaphore()
    for nbr in (left, right):
        pl.semaphore_signal(
            barrier_sem, inc=1,
            device_id=(nbr,), device_id_type=pl.DeviceIdType.MESH,
        )
    pl.semaphore_wait(barrier_sem, 2)

    local = pltpu.make_async_copy(
        x_ref, out_ref.at[pl.ds(my * m_per, m_per)], copy_sem
    )
    local.start()

    for h in range(N_DEV - 1):
        origin = (my - h) % N_DEV
        src = x_ref if h == 0 else out_ref.at[pl.ds(origin * m_per, m_per)]
        rdma = pltpu.make_async_remote_copy(
            src_ref=src,
            dst_ref=out_ref.at[pl.ds(origin * m_per, m_per)],
            send_sem=send_sems.at[h],
            recv_sem=recv_sems.at[h],
            device_id=(right,),
            device_id_type=pl.DeviceIdType.MESH,
        )
        rdma.start()
        rdma.wait()

    local.wait()


def _all_gather(x_shard):
    m_per, k = x_shard.shape
    return pl.pallas_call(
        _ag_body,
        out_shape=jax.ShapeDtypeStruct((N_DEV * m_per, k), x_shard.dtype),
        in_specs=[pl.BlockSpec(memory_space=pltpu.ANY)],
        out_specs=pl.BlockSpec(memory_space=pltpu.ANY),
        scratch_shapes=[
            pltpu.SemaphoreType.DMA,
            pltpu.SemaphoreType.DMA((N_DEV - 1,)),
            pltpu.SemaphoreType.DMA((N_DEV - 1,)),
        ],
        compiler_params=pltpu.CompilerParams(collective_id=0),
    )(x_shard)


_BM = 1024
_BK = 2048


def _mm_body(x_ref, w_ref, o_ref, acc_ref, *, nk):
    @pl.when(pl.program_id(1) == 0)
    def _():
        acc_ref[...] = jnp.zeros_like(acc_ref)

    acc_ref[...] += jnp.dot(
        x_ref[...], w_ref[...], preferred_element_type=jnp.float32
    )

    @pl.when(pl.program_id(1) == nk - 1)
    def _():
        o_ref[...] = jnp.maximum(acc_ref[...], 0.0)


def _matmul_relu(x_full, w_shard):
    m, k = x_full.shape
    _, n = w_shard.shape
    nk = k // _BK
    import functools
    return pl.pallas_call(
        functools.partial(_mm_body, nk=nk),
        grid=(m // _BM, nk),
        in_specs=[
            pl.BlockSpec((_BM, _BK), lambda i, j: (i, j)),
            pl.BlockSpec((_BK, n), lambda i, j: (j, 0)),
        ],
        out_specs=pl.BlockSpec((_BM, n), lambda i, j: (i, 0)),
        out_shape=jax.ShapeDtypeStruct((m, n), jnp.float32),
        scratch_shapes=[pltpu.VMEM((_BM, n), jnp.float32)],
        compiler_params=pltpu.CompilerParams(
            dimension_semantics=("parallel", "arbitrary"),
        ),
    )(x_full, w_shard)


def kernel(x, w_mat):
    x_full = _all_gather(x)
    return _matmul_relu(x_full, w_mat)


# baseline (device time: 2421320 ns/iter reference)
import jax
import jax.numpy as jnp
from jax import lax
from jax.experimental import pallas as pl
from jax.experimental.pallas import tpu as pltpu

N_DEV = 4


def _ag_body(x_ref, out_ref, copy_sem, send_sems, recv_sems):
    my = lax.axis_index("i")
    left = (my - 1) % N_DEV
    right = (my + 1) % N_DEV
    m_per = x_ref.shape[0]

    barrier_sem = pltpu.get_barrier_semaphore()
    for nbr in (left, right):
        pl.semaphore_signal(
            barrier_sem, inc=1,
            device_id=(nbr,), device_id_type=pl.DeviceIdType.MESH,
        )
    pl.semaphore_wait(barrier_sem, 2)

    local = pltpu.make_async_copy(
        x_ref, out_ref.at[pl.ds(my * m_per, m_per)], copy_sem
    )
    local.start()

    for h in range(N_DEV - 1):
        origin = (my - h) % N_DEV
        src = x_ref if h == 0 else out_ref.at[pl.ds(origin * m_per, m_per)]
        rdma = pltpu.make_async_remote_copy(
            src_ref=src,
            dst_ref=out_ref.at[pl.ds(origin * m_per, m_per)],
            send_sem=send_sems.at[h],
            recv_sem=recv_sems.at[h],
            device_id=(right,),
            device_id_type=pl.DeviceIdType.MESH,
        )
        rdma.start()
        rdma.wait()

    local.wait()


def _all_gather(x_shard):
    m_per, k = x_shard.shape
    return pl.pallas_call(
        _ag_body,
        out_shape=jax.ShapeDtypeStruct((N_DEV * m_per, k), x_shard.dtype),
        in_specs=[pl.BlockSpec(memory_space=pltpu.MemorySpace.HBM)],
        out_specs=pl.BlockSpec(memory_space=pltpu.MemorySpace.HBM),
        scratch_shapes=[
            pltpu.SemaphoreType.DMA,
            pltpu.SemaphoreType.DMA((N_DEV - 1,)),
            pltpu.SemaphoreType.DMA((N_DEV - 1,)),
        ],
        compiler_params=pltpu.CompilerParams(collective_id=0),
    )(x_shard)


_BM = 512
_BK = 2048


def _mm_body(x_ref, w_ref, o_ref, acc_ref, *, nk):
    @pl.when(pl.program_id(1) == 0)
    def _():
        acc_ref[...] = jnp.zeros_like(acc_ref)

    acc_ref[...] += jnp.dot(
        x_ref[...], w_ref[...], preferred_element_type=jnp.float32
    )

    @pl.when(pl.program_id(1) == nk - 1)
    def _():
        o_ref[...] = jnp.maximum(acc_ref[...], 0.0)


def _matmul_relu(x_full, w_shard):
    m, k = x_full.shape
    _, n = w_shard.shape
    nk = k // _BK
    import functools
    return pl.pallas_call(
        functools.partial(_mm_body, nk=nk),
        grid=(m // _BM, nk),
        in_specs=[
            pl.BlockSpec((_BM, _BK), lambda i, j: (i, j)),
            pl.BlockSpec((_BK, n), lambda i, j: (j, 0)),
        ],
        out_specs=pl.BlockSpec((_BM, n), lambda i, j: (i, 0)),
        out_shape=jax.ShapeDtypeStruct((m, n), jnp.float32),
        scratch_shapes=[pltpu.VMEM((_BM, n), jnp.float32)],
        compiler_params=pltpu.CompilerParams(
            dimension_semantics=("parallel", "arbitrary"),
            vmem_limit_bytes=100 * 1024 * 1024,
        ),
    )(x_full, w_shard)


def kernel(x, w_mat):
    x_full = _all_gather(x)
    return _matmul_relu(x_full, w_mat)


# device time: 2297949 ns/iter; 1.0537x vs baseline; 1.0537x over previous
import jax
import jax.numpy as jnp
from jax import lax
from jax.experimental import pallas as pl
from jax.experimental.pallas import tpu as pltpu

N_DEV = 4


def _ag_body(x_ref, out_ref, copy_sem, cw_send, cw_recv, ccw_send, ccw_recv):
    my = lax.axis_index("i")
    left = (my - 1) % N_DEV
    right = (my + 1) % N_DEV
    m_per = x_ref.shape[0]
    mh = m_per // 2

    barrier_sem = pltpu.get_barrier_semaphore()
    for nbr in (left, right):
        pl.semaphore_signal(
            barrier_sem, inc=1,
            device_id=(nbr,), device_id_type=pl.DeviceIdType.MESH,
        )
    pl.semaphore_wait(barrier_sem, 2)

    local = pltpu.make_async_copy(
        x_ref, out_ref.at[pl.ds(my * m_per, m_per)], copy_sem
    )
    local.start()

    for h in range(N_DEV - 1):
        cw_origin = (my - h) % N_DEV
        ccw_origin = (my + h) % N_DEV
        cw_src = (
            x_ref.at[pl.ds(0, mh)]
            if h == 0
            else out_ref.at[pl.ds(cw_origin * m_per, mh)]
        )
        ccw_src = (
            x_ref.at[pl.ds(mh, mh)]
            if h == 0
            else out_ref.at[pl.ds(ccw_origin * m_per + mh, mh)]
        )
        cw = pltpu.make_async_remote_copy(
            src_ref=cw_src,
            dst_ref=out_ref.at[pl.ds(cw_origin * m_per, mh)],
            send_sem=cw_send.at[h],
            recv_sem=cw_recv.at[h],
            device_id=(right,),
            device_id_type=pl.DeviceIdType.MESH,
        )
        ccw = pltpu.make_async_remote_copy(
            src_ref=ccw_src,
            dst_ref=out_ref.at[pl.ds(ccw_origin * m_per + mh, mh)],
            send_sem=ccw_send.at[h],
            recv_sem=ccw_recv.at[h],
            device_id=(left,),
            device_id_type=pl.DeviceIdType.MESH,
        )
        cw.start()
        ccw.start()
        cw.wait()
        ccw.wait()

    local.wait()


def _all_gather(x_shard):
    m_per, k = x_shard.shape
    return pl.pallas_call(
        _ag_body,
        out_shape=jax.ShapeDtypeStruct((N_DEV * m_per, k), x_shard.dtype),
        in_specs=[pl.BlockSpec(memory_space=pltpu.MemorySpace.HBM)],
        out_specs=pl.BlockSpec(memory_space=pltpu.MemorySpace.HBM),
        scratch_shapes=[
            pltpu.SemaphoreType.DMA,
            pltpu.SemaphoreType.DMA((N_DEV - 1,)),
            pltpu.SemaphoreType.DMA((N_DEV - 1,)),
            pltpu.SemaphoreType.DMA((N_DEV - 1,)),
            pltpu.SemaphoreType.DMA((N_DEV - 1,)),
        ],
        compiler_params=pltpu.CompilerParams(collective_id=0),
    )(x_shard)


_BM = 512
_BK = 2048


def _mm_body(x_ref, w_ref, o_ref, acc_ref, *, nk):
    @pl.when(pl.program_id(1) == 0)
    def _():
        acc_ref[...] = jnp.zeros_like(acc_ref)

    acc_ref[...] += jnp.dot(
        x_ref[...], w_ref[...], preferred_element_type=jnp.float32
    )

    @pl.when(pl.program_id(1) == nk - 1)
    def _():
        o_ref[...] = jnp.maximum(acc_ref[...], 0.0)


def _matmul_relu(x_full, w_shard):
    m, k = x_full.shape
    _, n = w_shard.shape
    nk = k // _BK
    import functools
    return pl.pallas_call(
        functools.partial(_mm_body, nk=nk),
        grid=(m // _BM, nk),
        in_specs=[
            pl.BlockSpec((_BM, _BK), lambda i, j: (i, j)),
            pl.BlockSpec((_BK, n), lambda i, j: (j, 0)),
        ],
        out_specs=pl.BlockSpec((_BM, n), lambda i, j: (i, 0)),
        out_shape=jax.ShapeDtypeStruct((m, n), jnp.float32),
        scratch_shapes=[pltpu.VMEM((_BM, n), jnp.float32)],
        compiler_params=pltpu.CompilerParams(
            dimension_semantics=("parallel", "arbitrary"),
            vmem_limit_bytes=100 * 1024 * 1024,
        ),
    )(x_full, w_shard)


def kernel(x, w_mat):
    x_full = _all_gather(x)
    return _matmul_relu(x_full, w_mat)


# device time: 576353 ns/iter; 4.2011x vs baseline; 3.9871x over previous
import functools

import jax
import jax.numpy as jnp
from jax import lax
from jax.experimental import pallas as pl
from jax.experimental.pallas import tpu as pltpu

N_DEV = 4
BF = jnp.bfloat16


def _cast_body(src_ref, dst_ref):
    dst_ref[...] = src_ref[...].astype(dst_ref.dtype)


def _cast(a, dtype, bm=1024):
    m, n = a.shape
    return pl.pallas_call(
        _cast_body,
        grid=(m // bm,),
        in_specs=[pl.BlockSpec((bm, n), lambda i: (i, 0))],
        out_specs=pl.BlockSpec((bm, n), lambda i: (i, 0)),
        out_shape=jax.ShapeDtypeStruct((m, n), dtype),
    )(a)


def _agw_body(wb_ref, wf_ref, copy_sem, cw_send, cw_recv, ccw_send, ccw_recv):
    my = lax.axis_index("i")
    left = (my - 1) % N_DEV
    right = (my + 1) % N_DEV
    k, n_per = wb_ref.shape
    kh = k // 2

    barrier_sem = pltpu.get_barrier_semaphore()
    for nbr in (left, right):
        pl.semaphore_signal(
            barrier_sem, inc=1,
            device_id=(nbr,), device_id_type=pl.DeviceIdType.MESH,
        )
    pl.semaphore_wait(barrier_sem, 2)

    local = pltpu.make_async_copy(
        wb_ref, wf_ref.at[:, pl.ds(my * n_per, n_per)], copy_sem
    )
    local.start()

    for h in range(N_DEV - 1):
        cw_o = (my - h) % N_DEV
        ccw_o = (my + h) % N_DEV
        cw_src = (
            wb_ref.at[pl.ds(0, kh), :]
            if h == 0
            else wf_ref.at[pl.ds(0, kh), pl.ds(cw_o * n_per, n_per)]
        )
        ccw_src = (
            wb_ref.at[pl.ds(kh, kh), :]
            if h == 0
            else wf_ref.at[pl.ds(kh, kh), pl.ds(ccw_o * n_per, n_per)]
        )
        cw = pltpu.make_async_remote_copy(
            src_ref=cw_src,
            dst_ref=wf_ref.at[pl.ds(0, kh), pl.ds(cw_o * n_per, n_per)],
            send_sem=cw_send.at[h],
            recv_sem=cw_recv.at[h],
            device_id=(right,),
            device_id_type=pl.DeviceIdType.MESH,
        )
        ccw = pltpu.make_async_remote_copy(
            src_ref=ccw_src,
            dst_ref=wf_ref.at[pl.ds(kh, kh), pl.ds(ccw_o * n_per, n_per)],
            send_sem=ccw_send.at[h],
            recv_sem=ccw_recv.at[h],
            device_id=(left,),
            device_id_type=pl.DeviceIdType.MESH,
        )
        cw.start()
        ccw.start()
        cw.wait()
        ccw.wait()

    local.wait()


def _ag_w(w_b):
    k, n_per = w_b.shape
    return pl.pallas_call(
        _agw_body,
        out_shape=jax.ShapeDtypeStruct((k, N_DEV * n_per), w_b.dtype),
        in_specs=[pl.BlockSpec(memory_space=pltpu.MemorySpace.HBM)],
        out_specs=pl.BlockSpec(memory_space=pltpu.MemorySpace.HBM),
        scratch_shapes=[
            pltpu.SemaphoreType.DMA,
            pltpu.SemaphoreType.DMA((N_DEV - 1,)),
            pltpu.SemaphoreType.DMA((N_DEV - 1,)),
            pltpu.SemaphoreType.DMA((N_DEV - 1,)),
            pltpu.SemaphoreType.DMA((N_DEV - 1,)),
        ],
        compiler_params=pltpu.CompilerParams(collective_id=0),
    )(w_b)


_BM = 512
_BN = 2048
_BK = 1024


def _mm_body(x_ref, w_ref, y_ref, acc_ref, *, nk):
    @pl.when(pl.program_id(2) == 0)
    def _():
        acc_ref[...] = jnp.zeros_like(acc_ref)

    acc_ref[...] += jnp.dot(
        x_ref[...].astype(BF), w_ref[...], preferred_element_type=jnp.float32
    )

    @pl.when(pl.program_id(2) == nk - 1)
    def _():
        y_ref[...] = jnp.maximum(acc_ref[...], 0.0).astype(y_ref.dtype)


def _gemm_relu(x_shard, w_full):
    m, k = x_shard.shape
    _, n = w_full.shape
    nk = k // _BK
    return pl.pallas_call(
        functools.partial(_mm_body, nk=nk),
        grid=(m // _BM, n // _BN, nk),
        in_specs=[
            pl.BlockSpec((_BM, _BK), lambda i, j, q: (i, q)),
            pl.BlockSpec((_BK, _BN), lambda i, j, q: (q, j)),
        ],
        out_specs=pl.BlockSpec((_BM, _BN), lambda i, j, q: (i, j)),
        out_shape=jax.ShapeDtypeStruct((m, n), BF),
        scratch_shapes=[pltpu.VMEM((_BM, _BN), jnp.float32)],
        compiler_params=pltpu.CompilerParams(
            dimension_semantics=("parallel", "parallel", "arbitrary"),
            vmem_limit_bytes=100 * 1024 * 1024,
        ),
    )(x_shard, w_full)


def _a2a_body(y_ref, outb_ref, copy_sem, send_sems, recv_sems):
    my = lax.axis_index("i")
    m_loc, n_full = y_ref.shape
    n_per = n_full // N_DEV

    barrier_sem = pltpu.get_barrier_semaphore()
    for d in (1, 2, 3):
        pl.semaphore_signal(
            barrier_sem, inc=1,
            device_id=((my + d) % N_DEV,),
            device_id_type=pl.DeviceIdType.MESH,
        )
    pl.semaphore_wait(barrier_sem, 3)

    local = pltpu.make_async_copy(
        y_ref.at[:, pl.ds(my * n_per, n_per)],
        outb_ref.at[pl.ds(my * m_loc, m_loc), :],
        copy_sem,
    )
    local.start()

    sends = []
    for d in (1, 2, 3):
        tgt = (my + d) % N_DEV
        r = pltpu.make_async_remote_copy(
            src_ref=y_ref.at[:, pl.ds(tgt * n_per, n_per)],
            dst_ref=outb_ref.at[pl.ds(my * m_loc, m_loc), :],
            send_sem=send_sems.at[d - 1],
            recv_sem=recv_sems.at[d - 1],
            device_id=(tgt,),
            device_id_type=pl.DeviceIdType.MESH,
        )
        r.start()
        sends.append(r)
    for r in sends:
        r.wait_send()

    for d in (1, 2, 3):
        src_dev = (my - d) % N_DEV
        recv = pltpu.make_async_remote_copy(
            src_ref=y_ref.at[:, pl.ds(0, n_per)],
            dst_ref=outb_ref.at[pl.ds(src_dev * m_loc, m_loc), :],
            send_sem=send_sems.at[d - 1],
            recv_sem=recv_sems.at[d - 1],
            device_id=(src_dev,),
            device_id_type=pl.DeviceIdType.MESH,
        )
        recv.wait_recv()

    local.wait()


def _a2a(y_b):
    m_loc, n_full = y_b.shape
    n_per = n_full // N_DEV
    return pl.pallas_call(
        _a2a_body,
        out_shape=jax.ShapeDtypeStruct((N_DEV * m_loc, n_per), y_b.dtype),
        in_specs=[pl.BlockSpec(memory_space=pltpu.MemorySpace.HBM)],
        out_specs=pl.BlockSpec(memory_space=pltpu.MemorySpace.HBM),
        scratch_shapes=[
            pltpu.SemaphoreType.DMA,
            pltpu.SemaphoreType.DMA((N_DEV - 1,)),
            pltpu.SemaphoreType.DMA((N_DEV - 1,)),
        ],
        compiler_params=pltpu.CompilerParams(collective_id=1),
    )(y_b)


def kernel(x, w_mat):
    w_b = _cast(w_mat, BF)
    w_full = _ag_w(w_b)
    y_b = _gemm_relu(x, w_full)
    out_b = _a2a(y_b)
    return _cast(out_b, jnp.float32)
